# baseline (speedup 1.0000x reference)
import functools

import jax
import jax.numpy as jnp
from jax import lax
from jax.experimental import pallas as pl
from jax.experimental.pallas import tpu as pltpu
from jax.experimental.pallas import tpu_sc as plsc

D_MODEL = 64
N_TOKENS = 4096 * 200
NUM_CORES = 2
NUM_SUBCORES = 16
NUM_WORKERS = NUM_CORES * NUM_SUBCORES
ROWS_PER_WORKER = N_TOKENS // NUM_WORKERS
CHUNK = 512
STEPS = ROWS_PER_WORKER // CHUNK

_mesh = plsc.VectorSubcoreMesh(core_axis_name="c", subcore_axis_name="s")


@functools.partial(
    pl.kernel,
    mesh=_mesh,
    out_type=jax.ShapeDtypeStruct((N_TOKENS, D_MODEL), jnp.float32),
    compiler_params=pltpu.CompilerParams(use_tc_tiling_on_sc=False),
    scratch_types=[
        pltpu.VMEM((CHUNK,), jnp.int32),
        pltpu.VMEM((CHUNK, D_MODEL), jnp.float32),
        pltpu.SemaphoreType.DMA,
    ],
)
def _gather_kernel(idx_hbm, table_hbm, out_hbm, idx_v, rows_v, sem):
    wid = lax.axis_index("s") * NUM_CORES + lax.axis_index("c")
    base = wid * ROWS_PER_WORKER

    def body(g, carry):
        off = base + g * CHUNK
        pltpu.sync_copy(idx_hbm.at[pl.ds(off, CHUNK)], idx_v)
        pltpu.async_copy(table_hbm.at[idx_v], rows_v, sem).wait()
        pltpu.sync_copy(rows_v, out_hbm.at[pl.ds(off, CHUNK)])
        return carry

    lax.fori_loop(0, STEPS, body, 0)


def kernel(x, table):
    flat = x.reshape(-1).astype(jnp.int32)
    out = _gather_kernel(flat, table)
    return out.reshape(x.shape + (table.shape[1],))


# speedup vs baseline: 1.0456x; 1.0456x over previous
import functools

import jax
import jax.numpy as jnp
from jax import lax
from jax.experimental import pallas as pl
from jax.experimental.pallas import tpu as pltpu
from jax.experimental.pallas import tpu_sc as plsc

D_MODEL = 64
N_TOKENS = 4096 * 200
NUM_CORES = 2
NUM_SUBCORES = 16
NUM_WORKERS = NUM_CORES * NUM_SUBCORES
ROWS_PER_WORKER = N_TOKENS // NUM_WORKERS
CHUNK = 400
NBUF = 4
STEPS = ROWS_PER_WORKER // CHUNK
GROUPS = STEPS // NBUF

_mesh = plsc.VectorSubcoreMesh(core_axis_name="c", subcore_axis_name="s")


@functools.partial(
    pl.kernel,
    mesh=_mesh,
    out_type=jax.ShapeDtypeStruct((N_TOKENS, D_MODEL), jnp.float32),
    compiler_params=pltpu.CompilerParams(use_tc_tiling_on_sc=False),
    scratch_types=[
        pltpu.VMEM((ROWS_PER_WORKER,), jnp.int32),
        pltpu.VMEM((NBUF, CHUNK, D_MODEL), jnp.float32),
    ]
    + [pltpu.SemaphoreType.DMA] * (2 * NBUF),
)
def _gather_kernel(idx_hbm, table_hbm, out_hbm, idx_v, rows_v, *sems):
    gsems = sems[:NBUF]
    ssems = sems[NBUF:]
    wid = lax.axis_index("s") * NUM_CORES + lax.axis_index("c")
    base = wid * ROWS_PER_WORKER

    pltpu.sync_copy(idx_hbm.at[pl.ds(base, ROWS_PER_WORKER)], idx_v)

    def idx_slice(g):
        return idx_v.at[pl.ds(g * CHUNK, CHUNK)]

    def gather_start(g, b):
        pltpu.async_copy(table_hbm.at[idx_slice(g)], rows_v.at[b], gsems[b])

    def gather_wait(g, b):
        pltpu.make_async_copy(table_hbm.at[idx_slice(g)], rows_v.at[b],
                              gsems[b]).wait()

    def out_slice(g):
        return out_hbm.at[pl.ds(base + g * CHUNK, CHUNK)]

    def store_start(g, b):
        pltpu.async_copy(rows_v.at[b], out_slice(g), ssems[b])

    def store_wait(g, b):
        pltpu.make_async_copy(rows_v.at[b], out_slice(g), ssems[b]).wait()

    for k in range(NBUF - 1):
        gather_start(k, k)
    gather_start(NBUF - 1, NBUF - 1)
    gather_wait(0, 0)
    store_start(0, 0)
    for g in range(1, NBUF):
        store_wait(g - 1, g - 1)
        gather_start(g + NBUF - 1, (g + NBUF - 1) % NBUF)
        gather_wait(g, g)
        store_start(g, g)

    def body(i, carry):
        for b in range(NBUF):
            g = i * NBUF + b
            store_wait(g - 1, (b - 1) % NBUF)
            gather_start(g + NBUF - 1, (b - 1) % NBUF)
            gather_wait(g, b)
            store_start(g, b)
        return carry

    lax.fori_loop(1, GROUPS - 1, body, 0)

    for b in range(NBUF):
        g = (GROUPS - 1) * NBUF + b
        if g + NBUF - 1 < STEPS:
            store_wait(g - 1, (g - 1) % NBUF)
            gather_start(g + NBUF - 1, (g + NBUF - 1) % NBUF)
        gather_wait(g, b)
        store_start(g, b)
    for b in range(NBUF):
        g = (GROUPS - 1) * NBUF + b
        store_wait(g, b)


def kernel(x, table):
    flat = x.reshape(-1).astype(jnp.int32)
    out = _gather_kernel(flat, table)
    return out.reshape(x.shape + (table.shape[1],))
